# baseline (speedup 1.0000x reference)
import functools

import jax
import jax.numpy as jnp
from jax import lax
from jax.experimental import pallas as pl
from jax.experimental.pallas import tpu as pltpu
from jax.experimental.pallas import tpu_sc as plsc

NUM_EMB = 1000000
B_TOK = 4096 * 200
EMB_D = 64
CH = 128
K = 5
GROUP = CH * K
NW = 32
BPW = B_TOK // NW
NG = BPW // GROUP
IR_PW = BPW // CH

_mesh = plsc.VectorSubcoreMesh(core_axis_name="c", subcore_axis_name="s")


@functools.partial(
    pl.kernel,
    mesh=_mesh,
    out_type=jax.ShapeDtypeStruct((B_TOK, EMB_D), jnp.float32),
    scratch_types=[
        pltpu.VMEM((IR_PW, CH), jnp.int32),
        pltpu.VMEM((2, GROUP, EMB_D), jnp.float32),
        pltpu.SemaphoreType.DMA,
        pltpu.SemaphoreType.DMA,
        pltpu.SemaphoreType.DMA,
    ],
    compiler_params=pltpu.CompilerParams(use_tc_tiling_on_sc=False),
)
def _embed_sc(idx_hbm, table_hbm, out_hbm, idx_v, rows_v, gsem, osem0, osem1):
    wid = lax.axis_index("s") * 2 + lax.axis_index("c")
    row0 = wid * IR_PW
    osems = (osem0, osem1)

    def gather_copies(g, b):
        return [
            pltpu.make_async_copy(
                table_hbm.at[idx_v.at[g * K + j]],
                rows_v.at[b].at[pl.ds(j * CH, CH)],
                gsem,
            )
            for j in range(K)
        ]

    def out_copy(g, b):
        return pltpu.make_async_copy(
            rows_v.at[b],
            out_hbm.at[pl.ds((row0 + g * K) * CH, GROUP)],
            osems[b],
        )

    pltpu.sync_copy(idx_hbm.at[pl.ds(row0, IR_PW)], idx_v)
    for c in gather_copies(0, 0):
        c.start()

    def body(s, _):
        for half in range(2):
            b = half
            ob = 1 - b
            g = 2 * s + half
            for c in gather_copies(g, b):
                c.wait()
            out_copy(g, b).start()

            @pl.when(g >= 1)
            def _():
                out_copy(g - 1, ob).wait()

            @pl.when(g + 1 < NG)
            def _():
                for c in gather_copies(g + 1, ob):
                    c.start()

        return 0

    lax.fori_loop(0, NG // 2, body, 0)
    out_copy(NG - 1, (NG - 1) % 2).wait()


_T1_BN = 2048


def _t1_body(x_ref, o_ref):
    o_ref[...] = x_ref[...].T


def _table_transpose(w_t):
    grid = (pl.cdiv(NUM_EMB, _T1_BN),)
    return pl.pallas_call(
        _t1_body,
        grid=grid,
        in_specs=[pl.BlockSpec((EMB_D, _T1_BN), lambda i: (0, i))],
        out_specs=pl.BlockSpec((_T1_BN, EMB_D), lambda i: (i, 0)),
        out_shape=jax.ShapeDtypeStruct((NUM_EMB, EMB_D), jnp.float32),
    )(w_t)


_T2_BB = 512
_T2_BT = 8


def _t2_body(x_ref, o_ref):
    for t in range(_T2_BT):
        o_ref[t] = x_ref[:, t, :].T


def _rows_to_native(rows3):
    grid = (4096 // _T2_BB, 200 // _T2_BT)
    return pl.pallas_call(
        _t2_body,
        grid=grid,
        in_specs=[pl.BlockSpec((_T2_BB, _T2_BT, EMB_D), lambda b, t: (b, t, 0))],
        out_specs=pl.BlockSpec((_T2_BT, EMB_D, _T2_BB), lambda b, t: (t, 0, b)),
        out_shape=jax.ShapeDtypeStruct((200, EMB_D, 4096), jnp.float32),
    )(rows3)


def kernel(input_, weight):
    table = _table_transpose(weight.T)
    idx = input_.reshape(B_TOK // CH, CH).astype(jnp.int32)
    rows = _embed_sc(idx, table)
    out_t = _rows_to_native(rows.reshape(4096, 200, EMB_D))
    return out_t.transpose(2, 0, 1)


# speedup vs baseline: 1.2286x; 1.2286x over previous
import functools

import jax
import jax.numpy as jnp
from jax import lax
from jax.experimental import pallas as pl
from jax.experimental.pallas import tpu as pltpu
from jax.experimental.pallas import tpu_sc as plsc

NUM_EMB = 1000000
B_TOK = 4096 * 200
EMB_D = 64
HALF = 500224
CH = 128
K = 5
GROUP = CH * K
NW = 32
BPW = B_TOK // NW
NG = BPW // GROUP
IR_PW = BPW // CH

_mesh = plsc.VectorSubcoreMesh(core_axis_name="c", subcore_axis_name="s")


@functools.partial(
    pl.kernel,
    mesh=_mesh,
    out_type=jax.ShapeDtypeStruct((B_TOK, EMB_D), jnp.float32),
    scratch_types=[
        pltpu.VMEM((IR_PW, CH), jnp.int32),
        pltpu.VMEM((2, GROUP, EMB_D), jnp.float32),
        pltpu.SemaphoreType.DMA,
        pltpu.SemaphoreType.DMA,
        pltpu.SemaphoreType.DMA,
    ],
    compiler_params=pltpu.CompilerParams(use_tc_tiling_on_sc=False),
)
def _embed_sc(idx_hbm, table_hbm, out_hbm, idx_v, rows_v, gsem, osem0, osem1):
    wid = lax.axis_index("s") * 2 + lax.axis_index("c")
    row0 = wid * IR_PW
    osems = (osem0, osem1)

    def gather_copies(g, b):
        return [
            pltpu.make_async_copy(
                table_hbm.at[idx_v.at[g * K + j]],
                rows_v.at[b].at[pl.ds(j * CH, CH)],
                gsem,
            )
            for j in range(K)
        ]

    def out_copy(g, b):
        return pltpu.make_async_copy(
            rows_v.at[b],
            out_hbm.at[pl.ds((row0 + g * K) * CH, GROUP)],
            osems[b],
        )

    pltpu.sync_copy(idx_hbm.at[pl.ds(row0, IR_PW)], idx_v)
    for c in gather_copies(0, 0):
        c.start()

    def body(s, _):
        for half in range(2):
            b = half
            ob = 1 - b
            g = 2 * s + half
            for c in gather_copies(g, b):
                c.wait()
            out_copy(g, b).start()

            @pl.when(g >= 1)
            def _():
                out_copy(g - 1, ob).wait()

            @pl.when(g + 1 < NG)
            def _():
                for c in gather_copies(g + 1, ob):
                    c.start()

        return 0

    lax.fori_loop(0, NG // 2, body, 0)
    out_copy(NG - 1, (NG - 1) % 2).wait()


def _eye64():
    i = lax.broadcasted_iota(jnp.int32, (64, 64), 0)
    j = lax.broadcasted_iota(jnp.int32, (64, 64), 1)
    return jnp.where(i == j, 1.0, 0.0).astype(jnp.float32)


_T1_BN = 512


def _t1_body(a_ref, b_ref, o_ref):
    e = _eye64()
    dn = (((0,), (0,)), ((), ()))
    o_ref[:, 0:64] = lax.dot_general(a_ref[...], e, dn,
                                     preferred_element_type=jnp.float32)
    o_ref[:, 64:128] = lax.dot_general(b_ref[...], e, dn,
                                       preferred_element_type=jnp.float32)


def _table_pack(w_t):
    grid = (HALF // _T1_BN,)
    return pl.pallas_call(
        _t1_body,
        grid=grid,
        in_specs=[
            pl.BlockSpec((EMB_D, _T1_BN), lambda i: (0, i)),
            pl.BlockSpec((EMB_D, _T1_BN), lambda i: (0, i + HALF // _T1_BN)),
        ],
        out_specs=pl.BlockSpec((_T1_BN, 2 * EMB_D), lambda i: (i, 0)),
        out_shape=jax.ShapeDtypeStruct((HALF, 2 * EMB_D), jnp.float32),
    )(w_t, w_t)


def _t2_body(x_ref, o_ref):
    e = _eye64()
    dn = (((1,), (1,)), ((), ()))
    x = x_ref[0]
    o_ref[0, :, 0:2048] = lax.dot_general(e, x[:, 0:64], dn,
                                          preferred_element_type=jnp.float32)
    o_ref[0, :, 2048:4096] = lax.dot_general(e, x[:, 64:128], dn,
                                             preferred_element_type=jnp.float32)


def _rows_to_native(x3):
    grid = (200,)
    return pl.pallas_call(
        _t2_body,
        grid=grid,
        in_specs=[pl.BlockSpec((1, 2048, 128), lambda t: (t, 0, 0))],
        out_specs=pl.BlockSpec((1, EMB_D, 4096), lambda t: (t, 0, 0)),
        out_shape=jax.ShapeDtypeStruct((200, EMB_D, 4096), jnp.float32),
    )(x3)


def kernel(input_, weight):
    table = _table_pack(weight.T).reshape(2 * HALF, EMB_D)
    i4 = input_.T.reshape(200, 2, 2048).transpose(0, 2, 1)
    flat = i4.reshape(B_TOK).astype(jnp.int32)
    flat = jnp.where(flat < HALF, 2 * flat, 2 * (flat - HALF) + 1)
    rows = _embed_sc(flat.reshape(B_TOK // CH, CH), table)
    out_t = _rows_to_native(rows.reshape(200, 2048, 128))
    return out_t.transpose(2, 0, 1)


# speedup vs baseline: 2.6153x; 2.1286x over previous
import functools

import jax
import jax.numpy as jnp
from jax import lax
from jax.experimental import pallas as pl
from jax.experimental.pallas import tpu as pltpu
from jax.experimental.pallas import tpu_sc as plsc

NUM_EMB = 1000000
B_TOK = 4096 * 200
EMB_D = 64
HALF = 524288
NW = 32
T2G = 4
NGRP = 200 // T2G

_mesh = plsc.VectorSubcoreMesh(core_axis_name="c", subcore_axis_name="s")


@functools.partial(
    pl.kernel,
    mesh=_mesh,
    out_type=jax.ShapeDtypeStruct((B_TOK, EMB_D), jnp.float32),
    scratch_types=[
        pltpu.VMEM((200, 128), jnp.int32),
        pltpu.VMEM((2, T2G * 128, EMB_D), jnp.float32),
        pltpu.SemaphoreType.DMA,
        pltpu.SemaphoreType.DMA,
        pltpu.SemaphoreType.DMA,
    ],
    compiler_params=pltpu.CompilerParams(use_tc_tiling_on_sc=False),
)
def _embed_sc(idx_hbm, table_hbm, out_hbm, idx_v, rows_v, gsem, wsem0, wsem1):
    wid = lax.axis_index("s") * 2 + lax.axis_index("c")
    q0 = wid * 64
    wsems = (wsem0, wsem1)

    def gather_copies(g, b):
        return [
            pltpu.make_async_copy(
                table_hbm.at[idx_v.at[g * T2G + j]],
                rows_v.at[b].at[pl.ds(j * 128, 128)],
                gsem,
            )
            for j in range(T2G)
        ]

    def write_copies(g, b):
        return [
            pltpu.make_async_copy(
                rows_v.at[b].at[pl.ds(j * 128, 128)],
                out_hbm.at[pl.ds((g * T2G + j) * 4096 + 2 * q0, 128)],
                wsems[b],
            )
            for j in range(T2G)
        ]

    pltpu.sync_copy(idx_hbm.at[wid], idx_v)
    for c in gather_copies(0, 0):
        c.start()

    def body(s, _):
        for half in range(2):
            b = half
            ob = 1 - b
            g = 2 * s + half
            for c in gather_copies(g, b):
                c.wait()
            for c in write_copies(g, b):
                c.start()

            @pl.when(g >= 1)
            def _():
                for c in write_copies(g - 1, ob):
                    c.wait()

            @pl.when(g + 1 < NGRP)
            def _():
                for c in gather_copies(g + 1, ob):
                    c.start()

        return 0

    lax.fori_loop(0, NGRP // 2, body, 0)
    for c in write_copies(NGRP - 1, (NGRP - 1) % 2):
        c.wait()


def _eye64():
    i = lax.broadcasted_iota(jnp.int32, (64, 64), 0)
    j = lax.broadcasted_iota(jnp.int32, (64, 64), 1)
    return jnp.where(i == j, 1.0, 0.0).astype(jnp.float32)


_T1_BN = 8192


def _t1_body(a_ref, b_ref, o_ref):
    e = _eye64()
    dn = (((0,), (0,)), ((), ()))
    o_ref[:, 0:64] = lax.dot_general(a_ref[...], e, dn,
                                     preferred_element_type=jnp.float32)
    o_ref[:, 64:128] = lax.dot_general(b_ref[...], e, dn,
                                       preferred_element_type=jnp.float32)


def _table_pack(w_t):
    grid = (HALF // _T1_BN,)
    return pl.pallas_call(
        _t1_body,
        grid=grid,
        in_specs=[
            pl.BlockSpec((EMB_D, _T1_BN), lambda i: (0, i)),
            pl.BlockSpec(
                (EMB_D, _T1_BN),
                lambda i: (0, jnp.minimum(i + HALF // _T1_BN,
                                          NUM_EMB // _T1_BN)),
            ),
        ],
        out_specs=pl.BlockSpec((_T1_BN, 2 * EMB_D), lambda i: (i, 0)),
        out_shape=jax.ShapeDtypeStruct((HALF, 2 * EMB_D), jnp.float32),
    )(w_t, w_t)


def _t2_body(x_ref, o_ref):
    e = _eye64()
    dn = (((1,), (1,)), ((), ()))
    x = x_ref[0]
    o_ref[0, :, 0:2048] = lax.dot_general(e, x[:, 0:64], dn,
                                          preferred_element_type=jnp.float32)
    o_ref[0, :, 2048:4096] = lax.dot_general(e, x[:, 64:128], dn,
                                             preferred_element_type=jnp.float32)


def _rows_to_native(x3):
    grid = (200,)
    return pl.pallas_call(
        _t2_body,
        grid=grid,
        in_specs=[pl.BlockSpec((1, 2048, 128), lambda t: (t, 0, 0))],
        out_specs=pl.BlockSpec((1, EMB_D, 4096), lambda t: (t, 0, 0)),
        out_shape=jax.ShapeDtypeStruct((200, EMB_D, 4096), jnp.float32),
    )(x3)


def kernel(input_, weight):
    table = _table_pack(weight.T).reshape(2 * HALF, EMB_D)
    flat = input_.T.reshape(B_TOK).astype(jnp.int32)
    flat = jnp.where(flat < HALF, 2 * flat, 2 * (flat - HALF) + 1)
    idx3 = flat.reshape(200, 2, 32, 64).transpose(2, 0, 3, 1).reshape(32, 200, 128)
    rows = _embed_sc(idx3, table)
    out_t = _rows_to_native(rows.reshape(200, 2048, 128))
    return out_t.transpose(2, 0, 1)


# speedup vs baseline: 2.6891x; 1.0282x over previous
import functools

import jax
import jax.numpy as jnp
from jax import lax
from jax.experimental import pallas as pl
from jax.experimental.pallas import tpu as pltpu
from jax.experimental.pallas import tpu_sc as plsc

NUM_EMB = 1000000
B_TOK = 4096 * 200
EMB_D = 64
HALF = 524288
NW = 32
NPH = 4
TPP = 200 // NPH
T2G = 5
NGRP = TPP // T2G

_mesh = plsc.VectorSubcoreMesh(core_axis_name="c", subcore_axis_name="s")


def _make_sc_phase(ph):
    @functools.partial(
        pl.kernel,
        mesh=_mesh,
        out_type=jax.ShapeDtypeStruct((TPP * 4096, EMB_D), jnp.float32),
        scratch_types=[
            pltpu.VMEM((TPP, 128), jnp.int32),
            pltpu.VMEM((2, T2G * 128, EMB_D), jnp.float32),
            pltpu.SemaphoreType.DMA,
            pltpu.SemaphoreType.DMA,
            pltpu.SemaphoreType.DMA,
        ],
        compiler_params=pltpu.CompilerParams(use_tc_tiling_on_sc=False),
        name=f"embed_sc_ph{ph}",
    )
    def _embed_sc(idx_hbm, table_hbm, out_hbm, idx_v, rows_v, gsem, w0, w1):
        wid = lax.axis_index("s") * 2 + lax.axis_index("c")
        q0 = wid * 64
        wsems = (w0, w1)

        def gather_copies(g, b):
            return [
                pltpu.make_async_copy(
                    table_hbm.at[idx_v.at[g * T2G + j]],
                    rows_v.at[b].at[pl.ds(j * 128, 128)],
                    gsem,
                )
                for j in range(T2G)
            ]

        def write_copies(g, b):
            return [
                pltpu.make_async_copy(
                    rows_v.at[b].at[pl.ds(j * 128, 128)],
                    out_hbm.at[pl.ds((g * T2G + j) * 4096 + 2 * q0, 128)],
                    wsems[b],
                )
                for j in range(T2G)
            ]

        pltpu.sync_copy(idx_hbm.at[ph, wid], idx_v)
        for c in gather_copies(0, 0):
            c.start()

        def body(s, _):
            for half in range(2):
                b = half
                ob = 1 - b
                g = 2 * s + half
                for c in gather_copies(g, b):
                    c.wait()
                for c in write_copies(g, b):
                    c.start()

                @pl.when(g >= 1)
                def _():
                    for c in write_copies(g - 1, ob):
                        c.wait()

                @pl.when(g + 1 < NGRP)
                def _():
                    for c in gather_copies(g + 1, ob):
                        c.start()

            return 0

        lax.fori_loop(0, NGRP // 2, body, 0)
        for c in write_copies(NGRP - 1, (NGRP - 1) % 2):
            c.wait()

    return _embed_sc


_sc_phases = [_make_sc_phase(p) for p in range(NPH)]


def _eye64():
    i = lax.broadcasted_iota(jnp.int32, (64, 64), 0)
    j = lax.broadcasted_iota(jnp.int32, (64, 64), 1)
    return jnp.where(i == j, 1.0, 0.0).astype(jnp.float32)


_T1_BN = 8192


def _t1_body(a_ref, b_ref, o_ref):
    e = _eye64()
    dn = (((0,), (0,)), ((), ()))
    o_ref[:, 0:64] = lax.dot_general(a_ref[...], e, dn,
                                     preferred_element_type=jnp.float32)
    o_ref[:, 64:128] = lax.dot_general(b_ref[...], e, dn,
                                       preferred_element_type=jnp.float32)


def _table_pack(w_t):
    grid = (HALF // _T1_BN,)
    return pl.pallas_call(
        _t1_body,
        grid=grid,
        in_specs=[
            pl.BlockSpec((EMB_D, _T1_BN), lambda i: (0, i)),
            pl.BlockSpec(
                (EMB_D, _T1_BN),
                lambda i: (0, jnp.minimum(i + HALF // _T1_BN,
                                          NUM_EMB // _T1_BN)),
            ),
        ],
        out_specs=pl.BlockSpec((_T1_BN, 2 * EMB_D), lambda i: (i, 0)),
        out_shape=jax.ShapeDtypeStruct((HALF, 2 * EMB_D), jnp.float32),
    )(w_t, w_t)


def _t2_body(x_ref, _, o_ref):
    e = _eye64()
    dn = (((1,), (1,)), ((), ()))
    x = x_ref[0]
    o_ref[0, :, 0:2048] = lax.dot_general(e, x[:, 0:64], dn,
                                          preferred_element_type=jnp.float32)
    o_ref[0, :, 2048:4096] = lax.dot_general(e, x[:, 64:128], dn,
                                             preferred_element_type=jnp.float32)


def _rows_to_native(x3, out_full, ph):
    return pl.pallas_call(
        functools.partial(_t2_body),
        grid=(TPP,),
        in_specs=[
            pl.BlockSpec((1, 2048, 128), lambda t: (t, 0, 0)),
            pl.BlockSpec(memory_space=pl.ANY),
        ],
        out_specs=pl.BlockSpec((1, EMB_D, 4096),
                               lambda t, ph=ph: (t + ph * TPP, 0, 0)),
        out_shape=jax.ShapeDtypeStruct((200, EMB_D, 4096), jnp.float32),
        input_output_aliases={1: 0},
    )(x3, out_full)


def _t2_first(x3):
    return pl.pallas_call(
        lambda x_ref, o_ref: _t2_body(x_ref, None, o_ref),
        grid=(TPP,),
        in_specs=[pl.BlockSpec((1, 2048, 128), lambda t: (t, 0, 0))],
        out_specs=pl.BlockSpec((1, EMB_D, 4096), lambda t: (t, 0, 0)),
        out_shape=jax.ShapeDtypeStruct((200, EMB_D, 4096), jnp.float32),
    )(x3)


def kernel(input_, weight):
    table = _table_pack(weight.T).reshape(2 * HALF, EMB_D)
    flat = input_.T.reshape(B_TOK).astype(jnp.int32)
    flat = jnp.where(flat < HALF, 2 * flat, 2 * (flat - HALF) + 1)
    idx3 = (flat.reshape(NPH, TPP, 2, 32, 64)
            .transpose(0, 3, 1, 4, 2)
            .reshape(NPH, 32, TPP, 128))
    rows = [sc(idx3, table).reshape(TPP, 2048, 128)
            for sc in _sc_phases]
    out_t = _t2_first(rows[0])
    for p in range(1, NPH):
        out_t = _rows_to_native(rows[p], out_t, p)
    return out_t.transpose(2, 0, 1)
